# baseline (speedup 1.0000x reference)
import jax
import jax.numpy as jnp
from jax import lax
from jax.experimental import pallas as pl
from jax.experimental.pallas import tpu as pltpu
from jax.experimental.pallas import tpu_sc as plsc

B, D, H, W, M = 32, 4, 128, 128, 128
HW = H * W
L = 16
NC = 2
NS = 16


def _tile_body(flat_hbm, pack_hbm, tgt_hbm, out_hbm,
               pack_v, pred_v, part_v, tgt_v, sem, sem_in):
    c = lax.axis_index("c")
    s = lax.axis_index("s")
    wid = s * NC + c

    cp_pack = pltpu.async_copy(pack_hbm.at[wid], pack_v, sem_in)
    cp_tgt = pltpu.async_copy(tgt_hbm.at[wid], tgt_v, sem_in)
    cp_pack.wait()

    copies = [
        pltpu.async_copy(flat_hbm.at[pack_v.at[d]], pred_v.at[d], sem)
        for d in range(D)
    ]

    macc = jnp.zeros((L,), jnp.float32)
    mvs = []
    for ch in range(M // L):
        mv = pack_v[D, pl.ds(ch * L, L)].astype(jnp.float32)
        mvs.append(mv)
        macc = macc + mv

    cp_tgt.wait()

    acc = jnp.zeros((L,), jnp.float32)
    for d in range(D):
        copies[d].wait()
        for ch in range(M // L):
            mv = mvs[ch]
            p = pred_v[d, pl.ds(ch * L, L)]
            t = tgt_v[d, pl.ds(ch * L, L)]
            diff = (p - t) * mv
            a = jnp.abs(diff)
            acc = acc + jnp.where(a < 1.0, 0.5 * diff * diff, a - 0.5)

    part_v[0, pl.ds(0, L)] = acc
    part_v[1, pl.ds(0, L)] = macc
    pltpu.sync_copy(part_v, out_hbm.at[wid])


@jax.jit
def kernel(output, mask, ind, target):
    flat = output.reshape(B * D * HW)
    offs = (jnp.arange(B, dtype=jnp.int32) * (D * HW))[:, None, None] + \
           (jnp.arange(D, dtype=jnp.int32) * HW)[None, :, None]
    idxs = ind.astype(jnp.int32)[:, None, :] + offs
    pack = jnp.concatenate(
        [idxs, mask.astype(jnp.int32)[:, None, :]], axis=1)
    tgt_t = jnp.transpose(target, (0, 2, 1))
    mesh = plsc.VectorSubcoreMesh(core_axis_name="c", subcore_axis_name="s")
    parts = pl.kernel(
        _tile_body,
        out_type=jax.ShapeDtypeStruct((NC * NS, 2, L), jnp.float32),
        mesh=mesh,
        scratch_types=[
            pltpu.VMEM((D + 1, M), jnp.int32),
            pltpu.VMEM((D, M), jnp.float32),
            pltpu.VMEM((2, L), jnp.float32),
            pltpu.VMEM((D, M), jnp.float32),
            pltpu.SemaphoreType.DMA,
            pltpu.SemaphoreType.DMA,
        ],
    )(flat, pack, tgt_t)
    total = parts[:, 0, :].sum()
    num = parts[:, 1, :].sum()
    return total / (num + 0.0001)


# speedup vs baseline: 1.0497x; 1.0497x over previous
import jax
import jax.numpy as jnp
from jax import lax
from jax.experimental import pallas as pl
from jax.experimental.pallas import tpu as pltpu
from jax.experimental.pallas import tpu_sc as plsc

B, D, H, W, M = 32, 4, 128, 128, 128
HW = H * W
L = 16
NS = 16
BPT = B // NS


def _tile_body(flat_hbm, pack_hbm, tgt_hbm, out_hbm,
               pack_v, pred_v, part_v, tgt_v,
               sem_pack, sem_tgt, *gsems):
    s = lax.axis_index("s")

    cp_pack = pltpu.async_copy(pack_hbm.at[pl.ds(s * BPT, BPT)], pack_v,
                               sem_pack)
    cp_tgt = pltpu.async_copy(tgt_hbm.at[pl.ds(s * BPT, BPT)], tgt_v,
                              sem_tgt)
    cp_pack.wait()

    copies = [
        pltpu.async_copy(flat_hbm.at[pack_v.at[bi, d]], pred_v.at[bi, d],
                         gsems[bi * D + d])
        for bi in range(BPT)
        for d in range(D)
    ]

    macc = jnp.zeros((L,), jnp.float32)
    mvs = []
    for bi in range(BPT):
        for ch in range(M // L):
            mv = pack_v[bi, D, pl.ds(ch * L, L)].astype(jnp.float32)
            mvs.append(mv)
            macc = macc + mv

    cp_tgt.wait()

    acc = jnp.zeros((L,), jnp.float32)
    for bi in range(BPT):
        for d in range(D):
            copies[bi * D + d].wait()
            for ch in range(M // L):
                mv = mvs[bi * (M // L) + ch]
                p = pred_v[bi, d, pl.ds(ch * L, L)]
                t = tgt_v[bi, d, pl.ds(ch * L, L)]
                diff = (p - t) * mv
                a = jnp.abs(diff)
                acc = acc + jnp.where(a < 1.0, 0.5 * diff * diff, a - 0.5)

    part_v[0, pl.ds(0, L)] = acc
    part_v[1, pl.ds(0, L)] = macc
    pltpu.sync_copy(part_v, out_hbm.at[s])


@jax.jit
def kernel(output, mask, ind, target):
    flat = output.reshape(B * D * HW)
    offs = (jnp.arange(B, dtype=jnp.int32) * (D * HW))[:, None, None] + \
           (jnp.arange(D, dtype=jnp.int32) * HW)[None, :, None]
    idxs = ind.astype(jnp.int32)[:, None, :] + offs
    pack = jnp.concatenate(
        [idxs, mask.astype(jnp.int32)[:, None, :]], axis=1)
    tgt_t = jnp.transpose(target, (0, 2, 1))
    mesh = plsc.VectorSubcoreMesh(core_axis_name="c", subcore_axis_name="s",
                                  num_cores=1)
    parts = pl.kernel(
        _tile_body,
        out_type=jax.ShapeDtypeStruct((NS, 2, L), jnp.float32),
        mesh=mesh,
        scratch_types=[
            pltpu.VMEM((BPT, D + 1, M), jnp.int32),
            pltpu.VMEM((BPT, D, M), jnp.float32),
            pltpu.VMEM((2, L), jnp.float32),
            pltpu.VMEM((BPT, D, M), jnp.float32),
        ] + [pltpu.SemaphoreType.DMA] * (2 + BPT * D),
    )(flat, pack, tgt_t)
    total = parts[:, 0, :].sum()
    num = parts[:, 1, :].sum()
    return total / (num + 0.0001)
